# baseline (speedup 1.0000x reference)
import jax
import jax.numpy as jnp
from jax import lax
from jax.experimental import pallas as pl
from jax.experimental.pallas import tpu as pltpu
from jax.experimental.pallas import tpu_sc as plsc

VOCAB = 100000
DIM = 64
PDIM = 128
B = 4096
L = 50
PL = 56

_INFO = plsc.get_sparse_core_info()
_NC = _INFO.num_cores
_NS = _INFO.num_subcores
_NW = _NC * _NS
_SENT_W = B // _NW
_SCH = 4
_NCHUNK = _SENT_W // _SCH
_NBUF = 4
_CIDX = _SCH * PL


def _make_gather():
  mesh = plsc.VectorSubcoreMesh(core_axis_name="c", subcore_axis_name="s")

  @pl.kernel(
      out_type=jax.ShapeDtypeStruct((B, PL, PDIM), jnp.float32),
      mesh=mesh,
      compiler_params=pltpu.CompilerParams(use_tc_tiling_on_sc=False),
      scratch_types=[
          pltpu.VMEM((_SENT_W * PL,), jnp.int32),
          [pltpu.VMEM((_SCH, PL, PDIM), jnp.float32) for _ in range(_NBUF)],
          [pltpu.SemaphoreType.DMA for _ in range(_NBUF)],
          [pltpu.SemaphoreType.DMA for _ in range(_NBUF)],
      ],
  )
  def gather_kernel(table_hbm, idx_hbm, out_hbm, idx_v, bufs, gsems, osems):
    wid = lax.axis_index("s") * _NC + lax.axis_index("c")
    sent0 = wid * _SENT_W
    pltpu.sync_copy(idx_hbm.at[pl.ds(sent0 * PL, _SENT_W * PL)], idx_v)

    def g_start(c, b):
      for s in range(_SCH):
        off = pl.multiple_of(c * _CIDX + s * PL, 8)
        pltpu.async_copy(
            table_hbm.at[idx_v.at[pl.ds(off, PL)]], bufs[b].at[s], gsems[b]
        )

    def g_wait(c, b):
      for s in range(_SCH):
        off = pl.multiple_of(c * _CIDX + s * PL, 8)
        pltpu.make_async_copy(
            table_hbm.at[idx_v.at[pl.ds(off, PL)]], bufs[b].at[s], gsems[b]
        ).wait()

    def o_copy(c, b):
      pltpu.async_copy(
          bufs[b], out_hbm.at[pl.ds(sent0 + c * _SCH, _SCH)], osems[b]
      ).wait()

    for b in range(_NBUF):
      g_start(b, b)

    @pl.loop(0, _NCHUNK - _NBUF, step=_NBUF)
    def _ring(i):
      for b in range(_NBUF):
        c = i + b
        g_wait(c, b)
        o_copy(c, b)
        g_start(c + _NBUF, b)

    for b in range(_NBUF):
      c = _NCHUNK - _NBUF + b
      g_wait(c, b)
      o_copy(c, b)

  return gather_kernel


_gather = _make_gather()


def kernel(W, X, init_emb):
  table_p = jnp.pad(init_emb, ((0, 0), (0, PDIM - DIM)))
  idx = jnp.pad(X.astype(jnp.int32), ((0, 0), (0, PL - L))).reshape(-1)
  out_pad = _gather(table_p, idx)
  return out_pad[:, :L, :DIM]


# speedup vs baseline: 5.2194x; 5.2194x over previous
import jax
import jax.numpy as jnp
from jax import lax
from jax.experimental import pallas as pl
from jax.experimental.pallas import tpu as pltpu
from jax.experimental.pallas import tpu_sc as plsc

VOCAB = 100000
DIM = 64
B = 4096
L = 50

_INFO = plsc.get_sparse_core_info()
_NC = _INFO.num_cores
_NS = _INFO.num_subcores
_NW = _NC * _NS
_TOTAL = B * L
_PER_W = _TOTAL // _NW
_CHUNK = 800
_NCHUNK = _PER_W // _CHUNK
_NBUF = 2


def _make_gather():
  mesh = plsc.VectorSubcoreMesh(core_axis_name="c", subcore_axis_name="s")

  @pl.kernel(
      out_type=jax.ShapeDtypeStruct((_TOTAL, DIM), jnp.float32),
      mesh=mesh,
      compiler_params=pltpu.CompilerParams(use_tc_tiling_on_sc=False),
      scratch_types=[
          pltpu.VMEM((_PER_W,), jnp.int32),
          [pltpu.VMEM((_CHUNK, DIM), jnp.float32) for _ in range(_NBUF)],
          [pltpu.SemaphoreType.DMA for _ in range(_NBUF)],
          [pltpu.SemaphoreType.DMA for _ in range(_NBUF)],
      ],
  )
  def gather_kernel(table_hbm, idx_hbm, out_hbm, idx_v, bufs, gsems, osems):
    wid = lax.axis_index("s") * _NC + lax.axis_index("c")
    base = wid * _PER_W
    pltpu.sync_copy(idx_hbm.at[pl.ds(base, _PER_W)], idx_v)

    def g_start(c_off, b):
      pltpu.async_copy(
          table_hbm.at[idx_v.at[pl.ds(c_off, _CHUNK)]], bufs[b], gsems[b]
      )

    def g_wait(c_off, b):
      pltpu.make_async_copy(
          table_hbm.at[idx_v.at[pl.ds(c_off, _CHUNK)]], bufs[b], gsems[b]
      ).wait()

    for b in range(_NBUF):
      g_start(b * _CHUNK, b)

    @pl.loop(0, _NCHUNK - _NBUF, step=_NBUF)
    def _ring(i):
      for b in range(_NBUF):
        off = pl.multiple_of((i + b) * _CHUNK, 8)
        g_wait(off, b)
        pltpu.async_copy(
            bufs[b], out_hbm.at[pl.ds(base + off, _CHUNK)], osems[b]
        ).wait()
        g_start(off + _NBUF * _CHUNK, b)

    for b in range(_NBUF):
      off = (_NCHUNK - _NBUF + b) * _CHUNK
      g_wait(off, b)
      pltpu.async_copy(
          bufs[b], out_hbm.at[pl.ds(base + off, _CHUNK)], osems[b]
      ).wait()

  return gather_kernel


_gather = _make_gather()


def kernel(W, X, init_emb):
  idx = X.reshape(-1).astype(jnp.int32)
  out = _gather(init_emb, idx)
  return out.reshape(B, L, DIM)


# speedup vs baseline: 5.2564x; 1.0071x over previous
import jax
import jax.numpy as jnp
from jax import lax
from jax.experimental import pallas as pl
from jax.experimental.pallas import tpu as pltpu
from jax.experimental.pallas import tpu_sc as plsc

VOCAB = 100000
DIM = 64
B = 4096
L = 50

_INFO = plsc.get_sparse_core_info()
_NC = _INFO.num_cores
_NS = _INFO.num_subcores
_NW = _NC * _NS
_TOTAL = B * L
_PER_W = _TOTAL // _NW
_CHUNK = 400
_NCHUNK = _PER_W // _CHUNK
_NBUF = 4


def _make_gather():
  mesh = plsc.VectorSubcoreMesh(core_axis_name="c", subcore_axis_name="s")

  @pl.kernel(
      out_type=jax.ShapeDtypeStruct((_TOTAL, DIM), jnp.float32),
      mesh=mesh,
      compiler_params=pltpu.CompilerParams(use_tc_tiling_on_sc=False),
      scratch_types=[
          pltpu.VMEM((_PER_W,), jnp.int32),
          [pltpu.VMEM((_CHUNK, DIM), jnp.float32) for _ in range(_NBUF)],
          [pltpu.SemaphoreType.DMA for _ in range(_NBUF)],
          [pltpu.SemaphoreType.DMA for _ in range(_NBUF)],
      ],
  )
  def gather_kernel(table_hbm, idx_hbm, out_hbm, idx_v, bufs, gsems, osems):
    wid = lax.axis_index("s") * _NC + lax.axis_index("c")
    base = wid * _PER_W
    pltpu.sync_copy(idx_hbm.at[pl.ds(base, _PER_W)], idx_v)

    def g_start(c_off, b):
      pltpu.async_copy(
          table_hbm.at[idx_v.at[pl.ds(c_off, _CHUNK)]], bufs[b], gsems[b]
      )

    def g_wait(c_off, b):
      pltpu.make_async_copy(
          table_hbm.at[idx_v.at[pl.ds(c_off, _CHUNK)]], bufs[b], gsems[b]
      ).wait()

    for b in range(_NBUF):
      g_start(b * _CHUNK, b)

    @pl.loop(0, _NCHUNK - _NBUF, step=_NBUF)
    def _ring(i):
      for b in range(_NBUF):
        off = pl.multiple_of((i + b) * _CHUNK, 8)
        g_wait(off, b)
        pltpu.async_copy(
            bufs[b], out_hbm.at[pl.ds(base + off, _CHUNK)], osems[b]
        ).wait()
        g_start(off + _NBUF * _CHUNK, b)

    for b in range(_NBUF):
      off = (_NCHUNK - _NBUF + b) * _CHUNK
      g_wait(off, b)
      pltpu.async_copy(
          bufs[b], out_hbm.at[pl.ds(base + off, _CHUNK)], osems[b]
      ).wait()

  return gather_kernel


_gather = _make_gather()


def kernel(W, X, init_emb):
  idx = X.reshape(-1).astype(jnp.int32)
  out = _gather(init_emb, idx)
  return out.reshape(B, L, DIM)
